# Initial kernel scaffold; baseline (speedup 1.0000x reference)
import jax
import jax.numpy as jnp
from jax.experimental import pallas as pl


def kernel(x, W1, W2, W3, W4, W5, g1, b1, g2, b2, g3, b3, g4, b4, g5, b5):
    raise NotImplementedError("write your pallas kernel here")


# speedup vs baseline: 10.5462x; 10.5462x over previous
import functools

import jax
import jax.numpy as jnp
from jax import lax
from jax.experimental import pallas as pl
from jax.experimental.pallas import tpu as pltpu
from jax.experimental.pallas import tpu_sc as plsc

F32 = jnp.float32
KNN = 20
DP = 128
HIGH = lax.Precision.HIGHEST
DEF = lax.Precision.DEFAULT
NEG = float("-inf")


def _knn_body(N, R, K, feats_blk, feats_all, idx_ref):
    b = pl.program_id(0)
    xb = feats_blk[0]
    xa = feats_all[0]
    dot = lax.dot_general(xb, xa, (((1,), (1,)), ((), ())),
                          precision=DEF, preferred_element_type=F32)
    xa2 = xa * xa
    ones = jnp.ones((8, DP), F32)
    sqj = lax.dot_general(ones, xa2, (((1,), (1,)), ((), ())),
                          precision=HIGH, preferred_element_type=F32)[0:1]
    sqi = jnp.sum(xb * xb, axis=1, keepdims=True)
    S = 2.0 * dot - sqj - sqi
    iota = lax.broadcasted_iota(jnp.int32, (R, N), 1)
    base = b * N
    for t in range(K):
        m = jnp.max(S, axis=1, keepdims=True)
        cand = jnp.where(S >= m, iota, N)
        j = jnp.min(cand, axis=1, keepdims=True)
        idx_ref[0, 0, :, t:t + 1] = j + base
        S = jnp.where(iota == j, NEG, S)


def _knn(B, N, R, K):
    NB = N // R
    return pl.pallas_call(
        functools.partial(_knn_body, N, R, K),
        grid=(B, NB),
        in_specs=[
            pl.BlockSpec((1, R, DP), lambda b, nb: (b, nb, 0)),
            pl.BlockSpec((1, N, DP), lambda b, nb: (b, 0, 0)),
        ],
        out_specs=pl.BlockSpec((1, 1, R, K), lambda b, nb: (b, nb, 0, 0)),
        out_shape=jax.ShapeDtypeStruct((B, NB, R, K), jnp.int32),
    )


def _sc_gather(BN, K):
    info = plsc.get_sparse_core_info()
    NC, NS = info.num_cores, info.num_subcores
    NW = NC * NS
    PPW = BN // NW
    P = 8
    CH = PPW // P
    HALF = (P * K) // 2
    mesh = plsc.VectorSubcoreMesh(core_axis_name="c", subcore_axis_name="s")

    @functools.partial(
        pl.kernel,
        out_type=jax.ShapeDtypeStruct((BN * K, DP), F32),
        mesh=mesh,
        scratch_types=[
            pltpu.VMEM((2, 128), jnp.int32),
            pltpu.VMEM((2, 128), jnp.int32),
            pltpu.VMEM((2, P * K, DP), F32),
            pltpu.SemaphoreType.DMA,
            pltpu.SemaphoreType.DMA,
            pltpu.SemaphoreType.DMA,
        ],
    )
    def sc_kernel(x_hbm, idx_hbm, g_hbm, idx_a, idx_b, rows_v,
                  sem0, sem1, wsem):
        wid = lax.axis_index("s") * NC + lax.axis_index("c")
        base = wid * PPW
        sems = (sem0, sem1)

        def start(g, sl):
            e0 = (base + g * P) * K
            pltpu.sync_copy(idx_hbm.at[pl.ds(e0, 128)], idx_a.at[sl])
            pltpu.sync_copy(idx_hbm.at[pl.ds(e0 + HALF, 128)], idx_b.at[sl])
            pltpu.async_copy(x_hbm.at[idx_a.at[sl].at[pl.ds(0, HALF)]],
                             rows_v.at[sl].at[pl.ds(0, HALF)], sems[sl])
            pltpu.async_copy(x_hbm.at[idx_b.at[sl].at[pl.ds(0, HALF)]],
                             rows_v.at[sl].at[pl.ds(HALF, HALF)], sems[sl])

        def drain(g, sl):
            e0 = (base + g * P) * K
            pltpu.make_async_copy(x_hbm.at[idx_a.at[sl].at[pl.ds(0, HALF)]],
                                  rows_v.at[sl].at[pl.ds(0, HALF)],
                                  sems[sl]).wait()
            pltpu.make_async_copy(x_hbm.at[idx_b.at[sl].at[pl.ds(0, HALF)]],
                                  rows_v.at[sl].at[pl.ds(HALF, HALF)],
                                  sems[sl]).wait()
            pltpu.async_copy(rows_v.at[sl], g_hbm.at[pl.ds(e0, P * K)],
                             wsem).wait()

        start(0, 0)

        def pair(h, carry):
            g0 = 2 * h
            start(g0 + 1, 1)
            drain(g0, 0)

            @pl.when(g0 + 2 < CH)
            def _():
                start(g0 + 2, 0)

            drain(g0 + 1, 1)
            return carry

        lax.fori_loop(0, CH // 2, pair, 0)

    return sc_kernel


def _edge_body(C, R, K, gath_ref, feats_blk, wd, wx, m_ref, st_ref):
    xb = feats_blk[0]
    g = gath_ref[0]
    xrep = jnp.broadcast_to(xb[:, None, :], (R, K, DP)).reshape(R * K, DP)
    y = (lax.dot_general(g - xrep, wd[...], (((1,), (0,)), ((), ())),
                         precision=DEF, preferred_element_type=F32)
         + lax.dot_general(xrep, wx[...], (((1,), (0,)), ((), ())),
                           precision=DEF, preferred_element_type=F32))
    m_ref[0] = jnp.max(y.reshape(R, K, C), axis=1)
    st_ref[0, 0] = jnp.concatenate(
        [jnp.sum(y, axis=0, keepdims=True),
         jnp.sum(y * y, axis=0, keepdims=True)], axis=0)


def _edge(C, B, N, R, K):
    NB = N // R
    return pl.pallas_call(
        functools.partial(_edge_body, C, R, K),
        grid=(B, NB),
        in_specs=[
            pl.BlockSpec((1, R * K, DP), lambda b, nb: (b, nb, 0)),
            pl.BlockSpec((1, R, DP), lambda b, nb: (b, nb, 0)),
            pl.BlockSpec((DP, C), lambda b, nb: (0, 0)),
            pl.BlockSpec((DP, C), lambda b, nb: (0, 0)),
        ],
        out_specs=[
            pl.BlockSpec((1, R, C), lambda b, nb: (b, nb, 0)),
            pl.BlockSpec((1, 1, 2, C), lambda b, nb: (b, nb, 0, 0)),
        ],
        out_shape=[
            jax.ShapeDtypeStruct((B, N, C), F32),
            jax.ShapeDtypeStruct((B, NB, 2, C), F32),
        ],
    )


def _finish_body(C, CO, RF, ss_ref, m_ref, o_ref):
    ss = ss_ref[...]
    z = m_ref[...] * ss[0:1, :C] + ss[1:2, :C]
    z = jnp.where(z >= 0, z, 0.2 * z)
    if CO > C:
        z = jnp.concatenate([z, jnp.zeros((RF, CO - C), F32)], axis=1)
    o_ref[...] = z


def _finish(C, CO, BN, RF):
    return pl.pallas_call(
        functools.partial(_finish_body, C, CO, RF),
        grid=(BN // RF,),
        in_specs=[
            pl.BlockSpec((2, CO), lambda i: (0, 0)),
            pl.BlockSpec((RF, C), lambda i: (i, 0)),
        ],
        out_specs=pl.BlockSpec((RF, CO), lambda i: (i, 0)),
        out_shape=jax.ShapeDtypeStruct((BN, CO), F32),
    )


def _final_body(cs, R, o1, o2, o3, o4, w5t, ymax_ref, yst_ref):
    cat = jnp.concatenate(
        [o1[0][:, :cs[0]], o2[0][:, :cs[1]], o3[0][:, :cs[2]],
         o4[0][:, :cs[3]]], axis=1)
    y = jnp.dot(cat, w5t[...], precision=DEF, preferred_element_type=F32)
    ymax_ref[0, 0] = jnp.max(y, axis=0, keepdims=True)
    yst_ref[0, 0] = jnp.concatenate(
        [jnp.sum(y, axis=0, keepdims=True),
         jnp.sum(y * y, axis=0, keepdims=True)], axis=0)


def _final(B, N, R, cs, ws):
    NB = N // R
    CO = 512
    return pl.pallas_call(
        functools.partial(_final_body, cs, R),
        grid=(B, NB),
        in_specs=[
            pl.BlockSpec((1, R, ws[0]), lambda b, nb: (b, nb, 0)),
            pl.BlockSpec((1, R, ws[1]), lambda b, nb: (b, nb, 0)),
            pl.BlockSpec((1, R, ws[2]), lambda b, nb: (b, nb, 0)),
            pl.BlockSpec((1, R, ws[3]), lambda b, nb: (b, nb, 0)),
            pl.BlockSpec((CO, CO), lambda b, nb: (0, 0)),
        ],
        out_specs=[
            pl.BlockSpec((1, 1, 1, CO), lambda b, nb: (b, nb, 0, 0)),
            pl.BlockSpec((1, 1, 2, CO), lambda b, nb: (b, nb, 0, 0)),
        ],
        out_shape=[
            jax.ShapeDtypeStruct((B, NB, 1, CO), F32),
            jax.ShapeDtypeStruct((B, NB, 2, CO), F32),
        ],
    )


def _edgeconv_layer(feats_p, W, g, b, B, N, R):
    BN = B * N
    C = W.shape[0]
    d = W.shape[1] // 2
    idx = _knn(B, N, R, KNN)(feats_p, feats_p)
    idxf = jnp.concatenate(
        [idx.reshape(BN * KNN), jnp.zeros((128,), jnp.int32)])
    gath = _sc_gather(BN, KNN)(feats_p.reshape(BN, DP), idxf)
    wd = jnp.zeros((DP, C), F32).at[:d].set(jnp.transpose(W[:, :d]))
    wx = jnp.zeros((DP, C), F32).at[:d].set(jnp.transpose(W[:, d:]))
    M, st = _edge(C, B, N, R, KNN)(gath.reshape(B, N * KNN, DP), feats_p,
                                   wd, wx)
    cnt = float(BN * KNN)
    mean = jnp.sum(st[:, :, 0, :], axis=(0, 1)) / cnt
    var = jnp.sum(st[:, :, 1, :], axis=(0, 1)) / cnt - mean * mean
    scale = lax.rsqrt(var + 1e-5) * g
    shift = b - mean * scale
    if C < DP:
        scale = jnp.concatenate([scale, jnp.zeros((DP - C,), F32)])
        shift = jnp.concatenate([shift, jnp.zeros((DP - C,), F32)])
        ss = jnp.stack([scale, shift])
        o = _finish(C, DP, BN, 1024)(ss, M.reshape(BN, C))
    else:
        ss = jnp.stack([scale, shift])
        o = _finish(C, C, BN, 1024)(ss, M.reshape(BN, C))
    return o, C


def kernel(x, W1, W2, W3, W4, W5, g1, b1, g2, b2, g3, b3, g4, b4, g5, b5):
    B, N, din = x.shape
    BN = B * N
    R = 256
    feats_p = jnp.concatenate(
        [x, jnp.zeros((B, N, DP - din), F32)], axis=2)
    outs, cs = [], []
    for W, g, b in ((W1, g1, b1), (W2, g2, b2), (W3, g3, b3), (W4, g4, b4)):
        o, C = _edgeconv_layer(feats_p, W, g, b, B, N, R)
        feats_p = o.reshape(B, N, -1)
        outs.append(feats_p)
        cs.append(C)
    ymax, yst = _final(B, N, R, tuple(cs), tuple(o.shape[-1] for o in outs))(
        *outs, jnp.transpose(W5))
    ymax_b = jnp.max(ymax[:, :, 0, :], axis=1)
    sy = jnp.sum(yst[:, :, 0, :], axis=(0, 1))
    sy2 = jnp.sum(yst[:, :, 1, :], axis=(0, 1))
    mean = sy / BN
    var = sy2 / BN - mean * mean
    scale = lax.rsqrt(var + 1e-5) * g5
    z = (ymax_b - mean) * scale + b5
    out = jnp.where(z >= 0, z, 0.2 * z)
    return out[:, None, :]
